# baseline (speedup 1.0000x reference)
import jax
import jax.numpy as jnp
from jax import lax
from jax.experimental import pallas as pl
from jax.experimental.pallas import tpu as pltpu
from jax.experimental.pallas import tpu_sc as plsc

NC = 2
NS = 16
NW = NC * NS

BATCH = 16384
HIST = 50
B = BATCH * HIST
D = 32
B_PER_W = B // NW
BR_PER_W = BATCH // NW
CHUNK = 800
BR_PER_CHUNK = CHUNK // HIST
N_CHUNKS = B_PER_W // CHUNK


def _gather_body(table_hbm, idx_hbm, out_hbm,
                 idx0, idx1, rows0, rows1, shuf0, shuf1,
                 isem0, isem1, gsem0, gsem1, wsem0, wsem1):
    wid = lax.axis_index("s") * NC + lax.axis_index("c")
    base = wid * B_PER_W
    base_br = wid * BR_PER_W

    idxs = (idx0, idx1)
    rows = (rows0, rows1)
    shufs = (shuf0, shuf1)
    isems = (isem0, isem1)
    gsems = (gsem0, gsem1)
    wsems = (wsem0, wsem1)

    iota = lax.iota(jnp.int32, 16)
    row_iota = iota * HIST

    def idxcopy(i, b):
        src = idx_hbm.at[pl.ds(base + i * CHUNK, CHUNK)]
        return pltpu.async_copy(src, idxs[b], isems[b])

    def gather(i, b):
        src = table_hbm.at[idxs[b]]
        return pltpu.async_copy(src, rows[b], gsems[b])

    def shuffle(b):
        rowsb = rows[b]
        shufb = shufs[b]

        @plsc.parallel_loop(0, HIST, unroll=5)
        def hbody(h):
            ridx = row_iota + h
            for tr in range(D // 8):
                for sl in range(8):
                    d = tr * 8 + sl
                    col = jnp.full((16,), d, jnp.int32)
                    shufb[h, tr, sl, :] = plsc.load_gather(rowsb, [ridx, col])

    def wbdma(i, b):
        br0 = base_br + i * BR_PER_CHUNK
        tc = br0 // 128
        ln0 = lax.rem(br0, 128)
        dst = out_hbm.at[:, :, tc, :, pl.ds(ln0, BR_PER_CHUNK)]
        return pltpu.async_copy(shufs[b], dst, wsems[b])

    def body(gg, carry):
        i0 = gg * 2
        i1 = i0 + 1
        hi0 = idxcopy(i0, 0)
        hi1 = idxcopy(i1, 1)
        hi0.wait()
        hg0 = gather(i0, 0)
        hi1.wait()
        hg1 = gather(i1, 1)
        hg0.wait()
        shuffle(0)
        hw0 = wbdma(i0, 0)
        hg1.wait()
        shuffle(1)
        hw1 = wbdma(i1, 1)
        hw0.wait()
        hw1.wait()
        return carry

    lax.fori_loop(0, N_CHUNKS // 2, body, 0)


@jax.jit
def kernel(input, weight):
    idx = input.reshape(-1).astype(jnp.int32)
    mesh = plsc.VectorSubcoreMesh(core_axis_name="c", subcore_axis_name="s")
    out5d = pl.kernel(
        _gather_body,
        mesh=mesh,
        out_type=jax.ShapeDtypeStruct(
            (HIST, D // 8, BATCH // 128, 8, 128), jnp.float32
        ),
        scratch_types=[
            pltpu.VMEM((CHUNK,), jnp.int32),
            pltpu.VMEM((CHUNK,), jnp.int32),
            pltpu.VMEM((CHUNK, D), jnp.float32),
            pltpu.VMEM((CHUNK, D), jnp.float32),
            pltpu.VMEM((HIST, D // 8, 8, BR_PER_CHUNK), jnp.float32),
            pltpu.VMEM((HIST, D // 8, 8, BR_PER_CHUNK), jnp.float32),
            pltpu.SemaphoreType.DMA,
            pltpu.SemaphoreType.DMA,
            pltpu.SemaphoreType.DMA,
            pltpu.SemaphoreType.DMA,
            pltpu.SemaphoreType.DMA,
            pltpu.SemaphoreType.DMA,
        ],
        compiler_params=pltpu.CompilerParams(
            use_tc_tiling_on_sc=False, needs_layout_passes=False
        ),
    )(weight, idx)
    return out5d.transpose(2, 4, 0, 1, 3).reshape(BATCH, HIST, D)


# speedup vs baseline: 1.0455x; 1.0455x over previous
import jax
import jax.numpy as jnp
from jax import lax
from jax.experimental import pallas as pl
from jax.experimental.pallas import tpu as pltpu
from jax.experimental.pallas import tpu_sc as plsc

NC = 2
NS = 16
NW = NC * NS

BATCH = 16384
HIST = 50
B = BATCH * HIST
D = 32
B_PER_W = B // NW
BR_PER_W = BATCH // NW
CHUNK = 800
BR_PER_CHUNK = CHUNK // HIST
N_CHUNKS = B_PER_W // CHUNK


def _gather_body(table_hbm, idx_hbm, out_hbm,
                 idx0, idx1, rows0, rows1, shuf0, shuf1,
                 isem0, isem1, gsem0, gsem1, wsem0, wsem1):
    wid = lax.axis_index("s") * NC + lax.axis_index("c")
    base = wid * B_PER_W
    base_br = wid * BR_PER_W

    idxs = (idx0, idx1)
    rows = (rows0, rows1)
    shufs = (shuf0, shuf1)
    isems = (isem0, isem1)
    gsems = (gsem0, gsem1)
    wsems = (wsem0, wsem1)

    iota = lax.iota(jnp.int32, 16)
    row_iota = iota * HIST

    def idxcopy(i, b):
        src = idx_hbm.at[pl.ds(base + i * CHUNK, CHUNK)]
        return pltpu.async_copy(src, idxs[b], isems[b])

    def gather(i, b):
        src = table_hbm.at[idxs[b]]
        return pltpu.async_copy(src, rows[b], gsems[b])

    def shuffle(b):
        rowsb = rows[b]
        shufb = shufs[b]

        @plsc.parallel_loop(0, HIST, unroll=2)
        def hbody(h):
            ridx = row_iota + h
            for tr in range(D // 8):
                for sl in range(8):
                    d = tr * 8 + sl
                    col = jnp.full((16,), d, jnp.int32)
                    shufb[h, tr, sl, :] = plsc.load_gather(rowsb, [ridx, col])

    def wbdma(i, b):
        br0 = base_br + i * BR_PER_CHUNK
        tc = br0 // 128
        ln0 = lax.rem(br0, 128)
        dst = out_hbm.at[:, :, tc, :, pl.ds(ln0, BR_PER_CHUNK)]
        return pltpu.async_copy(shufs[b], dst, wsems[b])

    def body(gg, carry):
        i0 = gg * 2
        i1 = i0 + 1
        hi0 = idxcopy(i0, 0)
        hi1 = idxcopy(i1, 1)
        hi0.wait()
        hg0 = gather(i0, 0)
        hi1.wait()
        hg1 = gather(i1, 1)
        hg0.wait()
        shuffle(0)
        hw0 = wbdma(i0, 0)
        hg1.wait()
        shuffle(1)
        hw1 = wbdma(i1, 1)
        hw0.wait()
        hw1.wait()
        return carry

    lax.fori_loop(0, N_CHUNKS // 2, body, 0)


@jax.jit
def kernel(input, weight):
    idx = input.reshape(-1).astype(jnp.int32)
    mesh = plsc.VectorSubcoreMesh(core_axis_name="c", subcore_axis_name="s")
    out5d = pl.kernel(
        _gather_body,
        mesh=mesh,
        out_type=jax.ShapeDtypeStruct(
            (HIST, D // 8, BATCH // 128, 8, 128), jnp.float32
        ),
        scratch_types=[
            pltpu.VMEM((CHUNK,), jnp.int32),
            pltpu.VMEM((CHUNK,), jnp.int32),
            pltpu.VMEM((CHUNK, D), jnp.float32),
            pltpu.VMEM((CHUNK, D), jnp.float32),
            pltpu.VMEM((HIST, D // 8, 8, BR_PER_CHUNK), jnp.float32),
            pltpu.VMEM((HIST, D // 8, 8, BR_PER_CHUNK), jnp.float32),
            pltpu.SemaphoreType.DMA,
            pltpu.SemaphoreType.DMA,
            pltpu.SemaphoreType.DMA,
            pltpu.SemaphoreType.DMA,
            pltpu.SemaphoreType.DMA,
            pltpu.SemaphoreType.DMA,
        ],
        compiler_params=pltpu.CompilerParams(
            use_tc_tiling_on_sc=False, needs_layout_passes=False
        ),
    )(weight, idx)
    return out5d.transpose(2, 4, 0, 1, 3).reshape(BATCH, HIST, D)
